# Initial kernel scaffold; baseline (speedup 1.0000x reference)
import jax
import jax.numpy as jnp
from jax.experimental import pallas as pl


def kernel(router_input, hidden_states, router_w, w_gate, w_up, w_down):
    raise NotImplementedError("write your pallas kernel here")


# speedup vs baseline: 1.3975x; 1.3975x over previous
import functools

import jax
import jax.numpy as jnp
from jax.experimental import pallas as pl
from jax.experimental.pallas import tpu as pltpu

S, H, FF, E, TOPK = 2048, 768, 768, 8, 2
BS = 256


def _routing_body(ri_ref, rw_ref, logits_ref, cmb_ref):
    ri = ri_ref[...]
    rw = rw_ref[...]
    logits = jax.lax.dot_general(
        ri, rw, (((1,), (1,)), ((), ())), preferred_element_type=jnp.float32
    )
    logits_ref[...] = logits
    colid = jax.lax.broadcasted_iota(jnp.int32, (S, E), 1)
    m1 = jnp.max(logits, axis=1, keepdims=True)
    idx1 = jnp.min(jnp.where(logits == m1, colid, E), axis=1, keepdims=True)
    l2 = jnp.where(colid == idx1, -jnp.inf, logits)
    m2 = jnp.max(l2, axis=1, keepdims=True)
    idx2 = jnp.min(jnp.where(l2 == m2, colid, E), axis=1, keepdims=True)
    e2 = jnp.exp(m2 - m1)
    denom = 1.0 + e2
    w1 = 1.0 / denom
    w2 = e2 / denom
    cmb_ref[...] = jnp.where(colid == idx1, w1, 0.0) + jnp.where(
        colid == idx2, w2, 0.0
    )


def _expert_body(x_ref, cmb_ref, wg_ref, wu_ref, wd_ref, out_ref):
    x = x_ref[...].astype(jnp.bfloat16)
    cmb = cmb_ref[...]
    acc = jnp.zeros((BS, H), jnp.float32)
    dims = (((1,), (0,)), ((), ()))
    for e in range(E):
        g = jax.lax.dot_general(
            x, wg_ref[e], dims, preferred_element_type=jnp.float32
        )
        u = jax.lax.dot_general(
            x, wu_ref[e], dims, preferred_element_type=jnp.float32
        )
        a = (jnp.maximum(g, 0.0) * u).astype(jnp.bfloat16)
        d = jax.lax.dot_general(
            a, wd_ref[e], dims, preferred_element_type=jnp.float32
        )
        acc = acc + cmb[:, e : e + 1] * d
    out_ref[...] = acc


@jax.jit
def kernel(router_input, hidden_states, router_w, w_gate, w_up, w_down):
    logits, cmb = pl.pallas_call(
        _routing_body,
        out_shape=(
            jax.ShapeDtypeStruct((S, E), jnp.float32),
            jax.ShapeDtypeStruct((S, E), jnp.float32),
        ),
    )(router_input, router_w)

    wg = w_gate.astype(jnp.bfloat16)
    wu = w_up.astype(jnp.bfloat16)
    wd = w_down.astype(jnp.bfloat16)

    grid = (S // BS,)
    out = pl.pallas_call(
        _expert_body,
        grid=grid,
        in_specs=[
            pl.BlockSpec((BS, H), lambda s: (s, 0)),
            pl.BlockSpec((BS, E), lambda s: (s, 0)),
            pl.BlockSpec((E, H, FF), lambda s: (0, 0, 0)),
            pl.BlockSpec((E, H, FF), lambda s: (0, 0, 0)),
            pl.BlockSpec((E, FF, H), lambda s: (0, 0, 0)),
        ],
        out_specs=pl.BlockSpec((BS, H), lambda s: (s, 0)),
        out_shape=jax.ShapeDtypeStruct((S, H), jnp.float32),
    )(hidden_states, cmb, wg, wu, wd)
    return (out, logits)
